# baseline (speedup 1.0000x reference)
import functools

import jax
import jax.numpy as jnp
from jax import lax
from jax.experimental import pallas as pl
from jax.experimental.pallas import tpu as pltpu
from jax.experimental.pallas import tpu_sc as plsc

N = 10000
E = 320000
D = 128
G = 64

NC = 2
NS = 16
NW = NC * NS
CH = 50
KB = 8
KG = 25
K = KB * KG
NBUF = 4

N_ACC = 10240
ROWS_PER_TILE = N_ACC // NS
ZCH = 16
RCH = 32


@functools.partial(
    pl.kernel,
    out_type=jax.ShapeDtypeStruct((NC, N_ACC, D), jnp.float32),
    mesh=plsc.VectorSubcoreMesh(core_axis_name="c", subcore_axis_name="s"),
    scratch_types=[
        pltpu.VMEM((3, KB, CH), jnp.int32),
        pltpu.VMEM((3, KB, CH), jnp.int32),
        pltpu.VMEM((NBUF, CH, D), jnp.float32),
        pltpu.VMEM_SHARED((N_ACC, D), jnp.float32),
        [pltpu.SemaphoreType.DMA] * NBUF,
        [pltpu.SemaphoreType.DMA] * NBUF,
        pltpu.SemaphoreType.DMA,
    ],
)
def _sc_scatter(hr_hbm, edges_hbm, out_hbm,
                src_v, dst_v, rows_v, acc_sh, gsem, ssem, zsem):
    cid = lax.axis_index("c")
    sid = lax.axis_index("s")
    wid = cid * NS + sid
    srcs_hbm = edges_hbm.at[0]
    dsts_hbm = edges_hbm.at[1]

    pltpu.sync_copy(srcs_hbm.at[wid].at[pl.ds(0, KB)], src_v.at[0])
    pltpu.sync_copy(dsts_hbm.at[wid].at[pl.ds(0, KB)], dst_v.at[0])
    for c in range(NBUF - 1):
        pltpu.async_copy(hr_hbm.at[src_v.at[0].at[c]], rows_v.at[c], gsem[c])

    zeros16 = jnp.zeros((16,), jnp.float32)
    for r in range(ZCH):
        for c8 in range(D // 16):
            rows_v[NBUF - 1, r, pl.ds(c8 * 16, 16)] = zeros16

    def _zero_body(j, carry):
        pltpu.async_copy(rows_v.at[NBUF - 1].at[pl.ds(0, ZCH)],
                         acc_sh.at[pl.ds(sid * ROWS_PER_TILE + j * ZCH, ZCH)],
                         zsem)
        return carry

    lax.fori_loop(0, ROWS_PER_TILE // ZCH, _zero_body, 0)

    def _zero_drain(j, carry):
        pltpu.make_async_copy(
            rows_v.at[NBUF - 1].at[pl.ds(0, ZCH)],
            acc_sh.at[pl.ds(sid * ROWS_PER_TILE + j * ZCH, ZCH)], zsem).wait()
        return carry

    lax.fori_loop(0, ROWS_PER_TILE // ZCH, _zero_drain, 0)
    plsc.subcore_barrier()

    def _group_body(g, carry):
        gb = g % 3

        @pl.when(g + 1 < KG)
        def _prefetch_idx():
            nb = (g + 1) % 3
            pltpu.sync_copy(srcs_hbm.at[wid].at[pl.ds((g + 1) * KB, KB)],
                            src_v.at[nb])
            pltpu.sync_copy(dsts_hbm.at[wid].at[pl.ds((g + 1) * KB, KB)],
                            dst_v.at[nb])

        for jj in range(KB):
            b = jj % NBUF
            pb = (jj - 1) % NBUF
            pltpu.make_async_copy(
                hr_hbm.at[src_v.at[gb].at[jj]], rows_v.at[b], gsem[b]).wait()
            pltpu.async_copy(
                rows_v.at[b], acc_sh.at[dst_v.at[gb].at[jj]], ssem[b],
                add=True)
            if jj == 0:
                @pl.when(g > 0)
                def _wait_prev():
                    pltpu.make_async_copy(
                        rows_v.at[pb],
                        acc_sh.at[dst_v.at[gb].at[jj]], ssem[pb]).wait()
            else:
                pltpu.make_async_copy(
                    rows_v.at[pb],
                    acc_sh.at[dst_v.at[gb].at[jj]], ssem[pb]).wait()
            nj = jj + NBUF - 1
            if nj < KB:
                pltpu.async_copy(hr_hbm.at[src_v.at[gb].at[nj]],
                                 rows_v.at[pb], gsem[pb])
            else:
                @pl.when(g + 1 < KG)
                def _next_group_gather():
                    pltpu.async_copy(
                        hr_hbm.at[src_v.at[(g + 1) % 3].at[nj - KB]],
                        rows_v.at[pb], gsem[pb])
        return carry

    lax.fori_loop(0, KG, _group_body, 0)
    pltpu.make_async_copy(
        rows_v.at[(K - 1) % NBUF], acc_sh.at[dst_v.at[0].at[0]],
        ssem[(K - 1) % NBUF]).wait()
    plsc.subcore_barrier()

    NRD = ROWS_PER_TILE // RCH
    for j in range(NRD):
        b = j % 2
        row = sid * ROWS_PER_TILE + j * RCH
        if j >= 2:
            prev = sid * ROWS_PER_TILE + (j - 2) * RCH
            pltpu.make_async_copy(
                rows_v.at[b].at[pl.ds(0, RCH)],
                out_hbm.at[cid].at[pl.ds(prev, RCH)], gsem[b]).wait()
        pltpu.sync_copy(acc_sh.at[pl.ds(row, RCH)],
                        rows_v.at[b].at[pl.ds(0, RCH)])
        pltpu.async_copy(rows_v.at[b].at[pl.ds(0, RCH)],
                         out_hbm.at[cid].at[pl.ds(row, RCH)], gsem[b])
    for j in range(max(NRD - 2, 0), NRD):
        b = j % 2
        row = sid * ROWS_PER_TILE + j * RCH
        pltpu.make_async_copy(
            rows_v.at[b].at[pl.ds(0, RCH)],
            out_hbm.at[cid].at[pl.ds(row, RCH)], gsem[b]).wait()


BN = 2000


def _layer_body(parts_ref, h_ref, wr_ref, ws_ref, b_ref, out_ref):
    agg = parts_ref[0] + parts_ref[1]
    out_ref[...] = jnp.tanh(
        jnp.dot(agg, wr_ref[...], preferred_element_type=jnp.float32)
        + b_ref[...]
        + jnp.dot(h_ref[...], ws_ref[...], preferred_element_type=jnp.float32)
    )


def _layer(parts, h, wr, ws, b2d):
    return pl.pallas_call(
        _layer_body,
        grid=(N // BN,),
        in_specs=[
            pl.BlockSpec((NC, BN, D), lambda i: (0, i, 0)),
            pl.BlockSpec((BN, D), lambda i: (i, 0)),
            pl.BlockSpec((D, D), lambda i: (0, 0)),
            pl.BlockSpec((D, D), lambda i: (0, 0)),
            pl.BlockSpec((1, D), lambda i: (0, 0)),
        ],
        out_specs=pl.BlockSpec((BN, D), lambda i: (i, 0)),
        out_shape=jax.ShapeDtypeStruct((N, D), jnp.float32),
    )(parts, h, wr, ws, b2d)


NP = 10112


def _pool_body(h_ref, batch_ref, wl_ref, bl_ref, out_ref):
    b = batch_ref[...]
    gids = lax.broadcasted_iota(jnp.int32, (G, NP), 0)
    mask = (b == gids).astype(jnp.float32)
    sums = jnp.dot(mask, h_ref[...], precision=lax.Precision.HIGHEST,
                   preferred_element_type=jnp.float32)
    counts = jnp.sum(mask, axis=1, keepdims=True)
    pooled = sums / jnp.maximum(counts, 1.0)
    out_ref[...] = (
        jnp.dot(pooled, wl_ref[...], preferred_element_type=jnp.float32)
        + bl_ref[...]
    )


def _pool(h3p, batch_p, wl, bl2d):
    return pl.pallas_call(
        _pool_body,
        in_specs=[
            pl.BlockSpec((NP, D), lambda: (0, 0)),
            pl.BlockSpec((1, NP), lambda: (0, 0)),
            pl.BlockSpec((D, 1), lambda: (0, 0)),
            pl.BlockSpec((1, 1), lambda: (0, 0)),
        ],
        out_specs=pl.BlockSpec((G, 1), lambda: (0, 0)),
        out_shape=jax.ShapeDtypeStruct((G, 1), jnp.float32),
    )(h3p, batch_p, wl, bl2d)


def kernel(x, edge_index, batch,
           W_rel0, W_root0, b0, W_rel1, W_root1, b1, W_rel2, W_root2, b2,
           W_lin, b_lin):
    edges = edge_index.reshape(2, NW, K, CH)

    h = x
    for (wr, ws, b) in ((W_rel0, W_root0, b0),
                        (W_rel1, W_root1, b1),
                        (W_rel2, W_root2, b2)):
        parts = _sc_scatter(h, edges)
        h = _layer(parts, h, wr, ws, b.reshape(1, D))

    h3p = jnp.pad(h, ((0, NP - N), (0, 0)))
    batch_p = jnp.pad(batch, (0, NP - N), constant_values=G).reshape(1, NP)
    return _pool(h3p, batch_p, W_lin, b_lin.reshape(1, 1))


# speedup vs baseline: 1.0880x; 1.0880x over previous
import functools

import jax
import jax.numpy as jnp
from jax import lax
from jax.experimental import pallas as pl
from jax.experimental.pallas import tpu as pltpu
from jax.experimental.pallas import tpu_sc as plsc

N = 10000
E = 320000
D = 128
G = 64

NC = 2
NS = 16
NW = NC * NS
CH = 64
KB = 16
KG = 10
K = KB * KG
E_PAD = NW * CH * K
NR = E // CH
NPAD = NW * K - NR
NBUF = 4

N_ACC = 10240
ROWS_PER_TILE = N_ACC // NS
ZCH = 16
RCH = CH


@functools.partial(
    pl.kernel,
    out_type=jax.ShapeDtypeStruct((NC, N_ACC, D), jnp.float32),
    mesh=plsc.VectorSubcoreMesh(core_axis_name="c", subcore_axis_name="s"),
    scratch_types=[
        pltpu.VMEM((3, KB, CH), jnp.int32),
        pltpu.VMEM((3, KB, CH), jnp.int32),
        pltpu.VMEM((NBUF, CH, D), jnp.float32),
        pltpu.VMEM_SHARED((N_ACC, D), jnp.float32),
        [pltpu.SemaphoreType.DMA] * NBUF,
        [pltpu.SemaphoreType.DMA] * NBUF,
        pltpu.SemaphoreType.DMA,
    ],
)
def _sc_scatter(hr_hbm, e3_hbm, pad_hbm, out_hbm,
                src_v, dst_v, rows_v, acc_sh, gsem, ssem, zsem):
    cid = lax.axis_index("c")
    sid = lax.axis_index("s")
    wid = cid * NS + sid

    def _load_idx_block(g, slot):
        rb = wid * K + g * KB
        for d in range(2):
            ev, pv = (src_v, 0) if d == 0 else (dst_v, 1)

            @pl.when(rb + KB <= NR)
            def _all_real():
                pltpu.sync_copy(e3_hbm.at[pv].at[pl.ds(rb, KB)], ev.at[slot])

            @pl.when(rb >= NR)
            def _all_pad():
                pltpu.sync_copy(pad_hbm.at[pv].at[pl.ds(rb - NR, KB)],
                                ev.at[slot])

            @pl.when(jnp.logical_and(rb < NR, rb + KB > NR))
            def _split():
                pltpu.sync_copy(e3_hbm.at[pv].at[pl.ds(rb, KB // 2)],
                                ev.at[slot].at[pl.ds(0, KB // 2)])
                pltpu.sync_copy(pad_hbm.at[pv].at[pl.ds(0, KB // 2)],
                                ev.at[slot].at[pl.ds(KB // 2, KB // 2)])

    _load_idx_block(0, 0)
    for c in range(NBUF - 1):
        pltpu.async_copy(hr_hbm.at[src_v.at[0].at[c]], rows_v.at[c], gsem[c])

    zeros16 = jnp.zeros((16,), jnp.float32)
    for r in range(ZCH):
        for c8 in range(D // 16):
            rows_v[NBUF - 1, r, pl.ds(c8 * 16, 16)] = zeros16

    def _zero_body(j, carry):
        pltpu.async_copy(rows_v.at[NBUF - 1].at[pl.ds(0, ZCH)],
                         acc_sh.at[pl.ds(sid * ROWS_PER_TILE + j * ZCH, ZCH)],
                         zsem)
        return carry

    lax.fori_loop(0, ROWS_PER_TILE // ZCH, _zero_body, 0)

    def _zero_drain(j, carry):
        pltpu.make_async_copy(
            rows_v.at[NBUF - 1].at[pl.ds(0, ZCH)],
            acc_sh.at[pl.ds(sid * ROWS_PER_TILE + j * ZCH, ZCH)], zsem).wait()
        return carry

    lax.fori_loop(0, ROWS_PER_TILE // ZCH, _zero_drain, 0)
    plsc.subcore_barrier()

    def _group_body(g, carry):
        gb = g % 3

        @pl.when(g + 1 < KG)
        def _prefetch_idx():
            _load_idx_block(g + 1, (g + 1) % 3)

        for jj in range(KB):
            b = jj % NBUF
            pb = (jj - 1) % NBUF
            pltpu.make_async_copy(
                hr_hbm.at[src_v.at[gb].at[jj]], rows_v.at[b], gsem[b]).wait()
            pltpu.async_copy(
                rows_v.at[b], acc_sh.at[dst_v.at[gb].at[jj]], ssem[b],
                add=True)
            if jj == 0:
                @pl.when(g > 0)
                def _wait_prev():
                    pltpu.make_async_copy(
                        rows_v.at[pb],
                        acc_sh.at[dst_v.at[gb].at[jj]], ssem[pb]).wait()
            else:
                pltpu.make_async_copy(
                    rows_v.at[pb],
                    acc_sh.at[dst_v.at[gb].at[jj]], ssem[pb]).wait()
            nj = jj + NBUF - 1
            if nj < KB:
                pltpu.async_copy(hr_hbm.at[src_v.at[gb].at[nj]],
                                 rows_v.at[pb], gsem[pb])
            else:
                @pl.when(g + 1 < KG)
                def _next_group_gather():
                    pltpu.async_copy(
                        hr_hbm.at[src_v.at[(g + 1) % 3].at[nj - KB]],
                        rows_v.at[pb], gsem[pb])
        return carry

    lax.fori_loop(0, KG, _group_body, 0)
    pltpu.make_async_copy(
        rows_v.at[(K - 1) % NBUF], acc_sh.at[dst_v.at[0].at[0]],
        ssem[(K - 1) % NBUF]).wait()
    plsc.subcore_barrier()

    NRD = ROWS_PER_TILE // RCH
    for j in range(NRD):
        b = j % 2
        row = sid * ROWS_PER_TILE + j * RCH
        if j >= 2:
            prev = sid * ROWS_PER_TILE + (j - 2) * RCH
            pltpu.make_async_copy(
                rows_v.at[b].at[pl.ds(0, RCH)],
                out_hbm.at[cid].at[pl.ds(prev, RCH)], gsem[b]).wait()
        pltpu.sync_copy(acc_sh.at[pl.ds(row, RCH)],
                        rows_v.at[b].at[pl.ds(0, RCH)])
        pltpu.async_copy(rows_v.at[b].at[pl.ds(0, RCH)],
                         out_hbm.at[cid].at[pl.ds(row, RCH)], gsem[b])
    for j in range(max(NRD - 2, 0), NRD):
        b = j % 2
        row = sid * ROWS_PER_TILE + j * RCH
        pltpu.make_async_copy(
            rows_v.at[b].at[pl.ds(0, RCH)],
            out_hbm.at[cid].at[pl.ds(row, RCH)], gsem[b]).wait()


BN = 2000


def _layer_body(parts_ref, h_ref, wr_ref, ws_ref, b_ref, out_ref):
    agg = parts_ref[0] + parts_ref[1]
    out_ref[...] = jnp.tanh(
        jnp.dot(agg, wr_ref[...], preferred_element_type=jnp.float32)
        + b_ref[...]
        + jnp.dot(h_ref[...], ws_ref[...], preferred_element_type=jnp.float32)
    )


def _layer(parts, h, wr, ws, b2d):
    return pl.pallas_call(
        _layer_body,
        grid=(N // BN,),
        in_specs=[
            pl.BlockSpec((NC, BN, D), lambda i: (0, i, 0)),
            pl.BlockSpec((BN, D), lambda i: (i, 0)),
            pl.BlockSpec((D, D), lambda i: (0, 0)),
            pl.BlockSpec((D, D), lambda i: (0, 0)),
            pl.BlockSpec((1, D), lambda i: (0, 0)),
        ],
        out_specs=pl.BlockSpec((BN, D), lambda i: (i, 0)),
        out_shape=jax.ShapeDtypeStruct((N, D), jnp.float32),
    )(parts, h, wr, ws, b2d)


NP = 10112


def _pool_body(h_ref, batch_ref, wl_ref, bl_ref, out_ref):
    b = batch_ref[...]
    gids = lax.broadcasted_iota(jnp.int32, (G, NP), 0)
    mask = (b == gids).astype(jnp.float32)
    sums = jnp.dot(mask, h_ref[...], precision=lax.Precision.HIGHEST,
                   preferred_element_type=jnp.float32)
    counts = jnp.sum(mask, axis=1, keepdims=True)
    pooled = sums / jnp.maximum(counts, 1.0)
    out_ref[...] = (
        jnp.dot(pooled, wl_ref[...], preferred_element_type=jnp.float32)
        + bl_ref[...]
    )


def _pool(h3p, batch_p, wl, bl2d):
    return pl.pallas_call(
        _pool_body,
        in_specs=[
            pl.BlockSpec((NP, D), lambda: (0, 0)),
            pl.BlockSpec((1, NP), lambda: (0, 0)),
            pl.BlockSpec((D, 1), lambda: (0, 0)),
            pl.BlockSpec((1, 1), lambda: (0, 0)),
        ],
        out_specs=pl.BlockSpec((G, 1), lambda: (0, 0)),
        out_shape=jax.ShapeDtypeStruct((G, 1), jnp.float32),
    )(h3p, batch_p, wl, bl2d)


def kernel(x, edge_index, batch,
           W_rel0, W_root0, b0, W_rel1, W_root1, b1, W_rel2, W_root2, b2,
           W_lin, b_lin):
    e3 = edge_index.reshape(2, NR, CH)
    pad_iota = jnp.arange(NPAD * CH, dtype=jnp.int32)
    pad_e = jnp.stack(
        [pad_iota % N, N + pad_iota % (N_ACC - N)]).reshape(2, NPAD, CH)

    h = x
    for (wr, ws, b) in ((W_rel0, W_root0, b0),
                        (W_rel1, W_root1, b1),
                        (W_rel2, W_root2, b2)):
        parts = _sc_scatter(h, e3, pad_e)
        h = _layer(parts, h, wr, ws, b.reshape(1, D))

    h3p = jnp.pad(h, ((0, NP - N), (0, 0)))
    batch_p = jnp.pad(batch, (0, NP - N), constant_values=G).reshape(1, NP)
    return _pool(h3p, batch_p, W_lin, b_lin.reshape(1, 1))
